# baseline (speedup 1.0000x reference)
import functools

import jax
import jax.numpy as jnp
from jax import lax
from jax.experimental import pallas as pl
from jax.experimental.pallas import tpu as pltpu
from jax.experimental.pallas import tpu_sc as plsc

NC, NS, L = 2, 16, 16
NW = NC * NS
B, T = 1024, 2048
F = 16384
STEP = F // T
ROWS = B // NW
SL = T // L

_mesh = plsc.VectorSubcoreMesh(
    core_axis_name="c", subcore_axis_name="s", num_cores=NC, num_subcores=NS
)


@functools.partial(
    pl.kernel,
    out_type=jax.ShapeDtypeStruct((B, F), jnp.float32),
    mesh=_mesh,
    scratch_types=[
        pltpu.VMEM((ROWS, T), jnp.float32),
        pltpu.VMEM((T,), jnp.float32),
        pltpu.VMEM((F,), jnp.float32),
        pltpu.VMEM((F,), jnp.float32),
        pltpu.SemaphoreType.DMA,
        pltpu.SemaphoreType.DMA,
    ],
    compiler_params=pltpu.CompilerParams(needs_layout_passes=False),
)
def _impulse(x_hbm, out_hbm, xall, erow, ob0, ob1, osem0, osem1):
    wid = lax.axis_index("s") * NC + lax.axis_index("c")
    base = wid * ROWS

    obufs = (ob0, ob1)
    osems = (osem0, osem1)

    pltpu.sync_copy(x_hbm.at[pl.ds(base, ROWS)], xall)

    zero16 = jnp.zeros((L,), jnp.float32)

    @pl.loop(0, F // L, unroll=8)
    def _(i):
        ob0[pl.ds(i * L, L)] = zero16
        ob1[pl.ds(i * L, L)] = zero16

    iota = lax.iota(jnp.int32, L)
    sidx = iota * STEP

    def shuffle(v, sh):
        return v.at[iota ^ sh].get(mode="promise_in_bounds")

    def allmax(v):
        for sh in (1, 2, 4, 8):
            v = jnp.maximum(v, shuffle(v, sh))
        return v

    def allsum(v):
        for sh in (1, 2, 4, 8):
            v = v + shuffle(v, sh)
        return v

    def do_row(r, b):
        ob = obufs[b]

        def maxbody(i, m):
            return jnp.maximum(m, xall[r, pl.ds(i * L, L)])

        m = lax.fori_loop(
            0, SL, maxbody, jnp.full((L,), -jnp.inf, jnp.float32), unroll=8
        )
        mm = allmax(m)

        def expbody(i, s):
            e = jnp.exp(xall[r, pl.ds(i * L, L)] - mm)
            erow[pl.ds(i * L, L)] = e
            return s + e

        s = lax.fori_loop(
            0, SL, expbody, jnp.zeros((L,), jnp.float32), unroll=8
        )
        recip = 1.0 / allsum(s)

        @pl.loop(0, SL, unroll=4)
        def _(i):
            v = erow[pl.ds(i * L, L)] * recip
            plsc.store_scatter(ob, [i * (L * STEP) + sidx], v)

        pltpu.make_async_copy(ob, out_hbm.at[base + r], osems[b]).start()

    do_row(0, 0)
    do_row(1, 1)

    @pl.loop(2, ROWS, step=2)
    def _(g):
        for b in range(2):
            r = g + b
            pltpu.make_async_copy(
                obufs[b], out_hbm.at[base + r - 2], osems[b]
            ).wait()
            do_row(r, b)

    pltpu.make_async_copy(ob0, out_hbm.at[base + ROWS - 2], osem0).wait()
    pltpu.make_async_copy(ob1, out_hbm.at[base + ROWS - 1], osem1).wait()


def kernel(x):
    return _impulse(x).reshape(B, 1, F)


# speedup vs baseline: 1.5226x; 1.5226x over previous
import functools

import jax
import jax.numpy as jnp
from jax import lax
from jax.experimental import pallas as pl
from jax.experimental.pallas import tpu as pltpu
from jax.experimental.pallas import tpu_sc as plsc

NC, NS, L = 2, 16, 16
NW = NC * NS
B, T = 1024, 2048
F = 16384
STEP = F // T
ROWS = B // NW
SL = T // L

_mesh = plsc.VectorSubcoreMesh(
    core_axis_name="c", subcore_axis_name="s", num_cores=NC, num_subcores=NS
)


@functools.partial(
    pl.kernel,
    out_type=jax.ShapeDtypeStruct((B, 1, F), jnp.float32),
    mesh=_mesh,
    scratch_types=[
        pltpu.VMEM((ROWS, T), jnp.float32),
        pltpu.VMEM((T,), jnp.float32),
        pltpu.VMEM((F,), jnp.float32),
        pltpu.VMEM((F,), jnp.float32),
        pltpu.SemaphoreType.DMA,
        pltpu.SemaphoreType.DMA,
    ],
    compiler_params=pltpu.CompilerParams(needs_layout_passes=False),
)
def _impulse(x_hbm, out_hbm, xall, erow, ob0, ob1, osem0, osem1):
    wid = lax.axis_index("s") * NC + lax.axis_index("c")
    base = wid * ROWS

    obufs = (ob0, ob1)
    osems = (osem0, osem1)

    pltpu.sync_copy(x_hbm.at[pl.ds(base, ROWS)], xall)

    zero16 = jnp.zeros((L,), jnp.float32)

    @pl.loop(0, F // L, unroll=8)
    def _(i):
        ob0[pl.ds(i * L, L)] = zero16
        ob1[pl.ds(i * L, L)] = zero16

    iota = lax.iota(jnp.int32, L)
    sidx = iota * STEP

    def shuffle(v, sh):
        return v.at[iota ^ sh].get(mode="promise_in_bounds")

    def allmax(v):
        for sh in (1, 2, 4, 8):
            v = jnp.maximum(v, shuffle(v, sh))
        return v

    def allsum(v):
        for sh in (1, 2, 4, 8):
            v = v + shuffle(v, sh)
        return v

    def do_row(r, b):
        ob = obufs[b]

        def maxbody(i, m):
            return jnp.maximum(m, xall[r, pl.ds(i * L, L)])

        m = lax.fori_loop(
            0, SL, maxbody, jnp.full((L,), -jnp.inf, jnp.float32), unroll=8
        )
        mm = allmax(m)

        def expbody(i, s):
            e = jnp.exp(xall[r, pl.ds(i * L, L)] - mm)
            erow[pl.ds(i * L, L)] = e
            return s + e

        s = lax.fori_loop(
            0, SL, expbody, jnp.zeros((L,), jnp.float32), unroll=8
        )
        recip = 1.0 / allsum(s)

        @pl.loop(0, SL, unroll=4)
        def _(i):
            v = erow[pl.ds(i * L, L)] * recip
            plsc.store_scatter(ob, [i * (L * STEP) + sidx], v)

        pltpu.make_async_copy(ob, out_hbm.at[base + r, 0], osems[b]).start()

    do_row(0, 0)
    do_row(1, 1)

    @pl.loop(2, ROWS, step=2)
    def _(g):
        for b in range(2):
            r = g + b
            pltpu.make_async_copy(
                obufs[b], out_hbm.at[base + r - 2, 0], osems[b]
            ).wait()
            do_row(r, b)

    pltpu.make_async_copy(ob0, out_hbm.at[base + ROWS - 2, 0], osem0).wait()
    pltpu.make_async_copy(ob1, out_hbm.at[base + ROWS - 1, 0], osem1).wait()


def kernel(x):
    return _impulse(x)


# speedup vs baseline: 3.1646x; 2.0784x over previous
import functools

import jax
import jax.numpy as jnp
from jax import lax
from jax.experimental import pallas as pl
from jax.experimental.pallas import tpu as pltpu
from jax.experimental.pallas import tpu_sc as plsc

NC, NS, L = 2, 16, 16
NW = NC * NS
B, T = 1024, 2048
F = 16384
STEP = F // T
ROWS = B // NW
SL = T // L

_mesh = plsc.VectorSubcoreMesh(
    core_axis_name="c", subcore_axis_name="s", num_cores=NC, num_subcores=NS
)


@functools.partial(
    pl.kernel,
    out_type=jax.ShapeDtypeStruct((B, 1, F), jnp.float32),
    mesh=_mesh,
    scratch_types=[
        pltpu.VMEM((ROWS, T), jnp.float32),
        pltpu.VMEM((F,), jnp.float32),
        pltpu.VMEM((F,), jnp.float32),
        pltpu.SemaphoreType.DMA,
        pltpu.SemaphoreType.DMA,
    ],
    compiler_params=pltpu.CompilerParams(needs_layout_passes=False),
)
def _impulse(x_hbm, out_hbm, xall, ob0, ob1, osem0, osem1):
    wid = lax.axis_index("s") * NC + lax.axis_index("c")
    base = wid * ROWS

    obufs = (ob0, ob1)
    osems = (osem0, osem1)

    pltpu.sync_copy(x_hbm.at[pl.ds(base, ROWS)], xall)

    zero16 = jnp.zeros((L,), jnp.float32)

    @plsc.parallel_loop(0, F // L, unroll=8)
    def _(i):
        ob0[pl.ds(i * L, L)] = zero16
        ob1[pl.ds(i * L, L)] = zero16

    iota = lax.iota(jnp.int32, L)
    sidx = iota * STEP

    def shuffle(v, sh):
        return v.at[iota ^ sh].get(mode="promise_in_bounds")

    def allmax(v):
        for sh in (1, 2, 4, 8):
            v = jnp.maximum(v, shuffle(v, sh))
        return v

    def allsum(v):
        for sh in (1, 2, 4, 8):
            v = v + shuffle(v, sh)
        return v

    def do_row(r, b):
        ob = obufs[b]

        @plsc.parallel_loop(
            0, SL, unroll=8, carry=jnp.full((L,), -jnp.inf, jnp.float32)
        )
        def mloop(i, m):
            return jnp.maximum(m, xall[r, pl.ds(i * L, L)])

        mm = allmax(mloop)

        @plsc.parallel_loop(
            0, SL, unroll=8, carry=jnp.zeros((L,), jnp.float32)
        )
        def sloop(i, s):
            return s + jnp.exp(xall[r, pl.ds(i * L, L)] - mm)

        recip = 1.0 / allsum(sloop)

        @plsc.parallel_loop(0, SL, unroll=8)
        def _(i):
            v = jnp.exp(xall[r, pl.ds(i * L, L)] - mm) * recip
            plsc.store_scatter(ob, [i * (L * STEP) + sidx], v)

        pltpu.make_async_copy(ob, out_hbm.at[base + r, 0], osems[b]).start()

    do_row(0, 0)
    do_row(1, 1)

    @pl.loop(2, ROWS, step=2)
    def _(g):
        for b in range(2):
            r = g + b
            pltpu.make_async_copy(
                obufs[b], out_hbm.at[base + r - 2, 0], osems[b]
            ).wait()
            do_row(r, b)

    pltpu.make_async_copy(ob0, out_hbm.at[base + ROWS - 2, 0], osem0).wait()
    pltpu.make_async_copy(ob1, out_hbm.at[base + ROWS - 1, 0], osem1).wait()


def kernel(x):
    return _impulse(x)


# speedup vs baseline: 3.2727x; 1.0342x over previous
import functools

import jax
import jax.numpy as jnp
from jax import lax
from jax.experimental import pallas as pl
from jax.experimental.pallas import tpu as pltpu
from jax.experimental.pallas import tpu_sc as plsc

NC, NS, L = 2, 16, 16
NW = NC * NS
B, T = 1024, 2048
F = 16384
STEP = F // T
ROWS = B // NW
SL = T // L

_mesh = plsc.VectorSubcoreMesh(
    core_axis_name="c", subcore_axis_name="s", num_cores=NC, num_subcores=NS
)


@functools.partial(
    pl.kernel,
    out_type=jax.ShapeDtypeStruct((B, 1, F), jnp.float32),
    mesh=_mesh,
    scratch_types=[
        pltpu.VMEM((ROWS, T), jnp.float32),
        pltpu.VMEM((F,), jnp.float32),
        pltpu.VMEM((F,), jnp.float32),
        pltpu.SemaphoreType.DMA,
        pltpu.SemaphoreType.DMA,
        pltpu.SemaphoreType.DMA,
    ],
    compiler_params=pltpu.CompilerParams(needs_layout_passes=False),
)
def _impulse(x_hbm, out_hbm, xall, ob0, ob1, osem0, osem1, isem):
    wid = lax.axis_index("s") * NC + lax.axis_index("c")
    base = wid * ROWS

    obufs = (ob0, ob1)
    osems = (osem0, osem1)

    incopy = pltpu.make_async_copy(x_hbm.at[pl.ds(base, ROWS)], xall, isem)
    incopy.start()

    zero16 = jnp.zeros((L,), jnp.float32)

    @plsc.parallel_loop(0, F // L, unroll=8)
    def _(i):
        ob0[pl.ds(i * L, L)] = zero16
        ob1[pl.ds(i * L, L)] = zero16

    incopy.wait()

    iota = lax.iota(jnp.int32, L)
    sidx = iota * STEP

    def shuffle(v, sh):
        return v.at[iota ^ sh].get(mode="promise_in_bounds")

    def allmax(v):
        for sh in (1, 2, 4, 8):
            v = jnp.maximum(v, shuffle(v, sh))
        return v

    def allsum(v):
        for sh in (1, 2, 4, 8):
            v = v + shuffle(v, sh)
        return v

    def do_row(r, b):
        ob = obufs[b]

        @plsc.parallel_loop(
            0, SL, unroll=8, carry=jnp.full((L,), -jnp.inf, jnp.float32)
        )
        def mloop(i, m):
            return jnp.maximum(m, xall[r, pl.ds(i * L, L)])

        mm = allmax(mloop)

        @plsc.parallel_loop(
            0, SL, unroll=8, carry=jnp.zeros((L,), jnp.float32)
        )
        def sloop(i, s):
            return s + jnp.exp(xall[r, pl.ds(i * L, L)] - mm)

        recip = 1.0 / allsum(sloop)

        @plsc.parallel_loop(0, SL, unroll=8)
        def _(i):
            v = jnp.exp(xall[r, pl.ds(i * L, L)] - mm) * recip
            plsc.store_scatter(ob, [i * (L * STEP) + sidx], v)

        pltpu.make_async_copy(ob, out_hbm.at[base + r, 0], osems[b]).start()

    do_row(0, 0)
    do_row(1, 1)

    @pl.loop(2, ROWS, step=2)
    def _(g):
        for b in range(2):
            r = g + b
            pltpu.make_async_copy(
                obufs[b], out_hbm.at[base + r - 2, 0], osems[b]
            ).wait()
            do_row(r, b)

    pltpu.make_async_copy(ob0, out_hbm.at[base + ROWS - 2, 0], osem0).wait()
    pltpu.make_async_copy(ob1, out_hbm.at[base + ROWS - 1, 0], osem1).wait()


def kernel(x):
    return _impulse(x)
